# baseline (speedup 1.0000x reference)
import jax
import jax.numpy as jnp
from jax import lax
from jax.experimental import pallas as pl

N = 8192
M = 2 * N
R = 128
C = 128
LOG_M = 14
NEG = float("-inf")
POS = float("inf")


def _xor_shuffle(z, j, r_iota, c_iota):
    if j < C:
        fwd = jnp.roll(z, -j, axis=1)
        bwd = jnp.roll(z, j, axis=1)
        return jnp.where((c_iota & j) == 0, fwd, bwd)
    jr = j // C
    fwd = jnp.roll(z, -jr, axis=0)
    bwd = jnp.roll(z, jr, axis=0)
    return jnp.where((r_iota & jr) == 0, fwd, bwd)


def _lane_scan(v, c_iota, op, fill):
    for s in (1, 2, 4, 8, 16, 32, 64):
        sh = jnp.roll(v, s, axis=1)
        sh = jnp.where(c_iota >= s, sh, fill)
        v = op(v, sh)
    return v


def _lane_scan_rev(v, c_iota, op, fill):
    for s in (1, 2, 4, 8, 16, 32, 64):
        sh = jnp.roll(v, -s, axis=1)
        sh = jnp.where(c_iota < C - s, sh, fill)
        v = op(v, sh)
    return v


def _row_scan_excl(col, r_iota1, op, fill):
    e = jnp.roll(col, 1, axis=0)
    e = jnp.where(r_iota1 >= 1, e, fill)
    for s in (1, 2, 4, 8, 16, 32, 64):
        sh = jnp.roll(e, s, axis=0)
        sh = jnp.where(r_iota1 >= s, sh, fill)
        e = op(e, sh)
    return e


def _row_scan_excl_rev(col, r_iota1, op, fill):
    e = jnp.roll(col, -1, axis=0)
    e = jnp.where(r_iota1 < R - 1, e, fill)
    for s in (1, 2, 4, 8, 16, 32, 64):
        sh = jnp.roll(e, -s, axis=0)
        sh = jnp.where(r_iota1 < R - s, sh, fill)
        e = op(e, sh)
    return e


def _cummax_incl(a, r_iota1, c_iota):
    v = _lane_scan(a, c_iota, jnp.maximum, NEG)
    rm = lax.slice(v, (0, C - 1), (R, C))
    e = _row_scan_excl(rm, r_iota1, jnp.maximum, NEG)
    return jnp.maximum(v, e)


def _revcummin_incl(a, r_iota1, c_iota):
    v = _lane_scan_rev(a, c_iota, jnp.minimum, POS)
    rm = lax.slice(v, (0, 0), (R, 1))
    e = _row_scan_excl_rev(rm, r_iota1, jnp.minimum, POS)
    return jnp.minimum(v, e)


def _chamfer_body(z_ref, out_ref):
    z = z_ref[...]
    r_iota = lax.broadcasted_iota(jnp.int32, (R, C), 0)
    c_iota = lax.broadcasted_iota(jnp.int32, (R, C), 1)
    r_iota1 = lax.broadcasted_iota(jnp.int32, (R, 1), 0)
    p = r_iota * C + c_iota

    zi = lax.bitcast_convert_type(z, jnp.int32)
    zi = jnp.where(p >= N, zi | 1, zi & jnp.int32(~1))
    z = lax.bitcast_convert_type(zi, jnp.float32)

    for kk in range(1, LOG_M + 1):
        kbit = 1 << kk
        for jj in range(kk - 1, -1, -1):
            j = 1 << jj
            partner = _xor_shuffle(z, j, r_iota, c_iota)
            wantmin = ((p & j) == 0) == ((p & kbit) == 0)
            mn = jnp.minimum(z, partner)
            mx = jnp.maximum(z, partner)
            z = jnp.where(wantmin, mn, mx)

    zi2 = lax.bitcast_convert_type(z, jnp.int32)
    is_y = (zi2 & 1) == 1

    ly = _cummax_incl(jnp.where(is_y, z, NEG), r_iota1, c_iota)
    ry = _revcummin_incl(jnp.where(is_y, z, POS), r_iota1, c_iota)
    dx = jnp.minimum(z - ly, ry - z)
    sum_x = jnp.sum(jnp.where(is_y, 0.0, dx))

    lx = _cummax_incl(jnp.where(is_y, NEG, z), r_iota1, c_iota)
    rx = _revcummin_incl(jnp.where(is_y, POS, z), r_iota1, c_iota)
    dy = jnp.minimum(z - lx, rx - z)
    sum_y = jnp.sum(jnp.where(is_y, dy, 0.0))

    loss = (0.5 / N) * sum_x + (0.5 / N) * sum_y
    out_ref[...] = jnp.full((1, 1), loss, dtype=jnp.float32)


def _kernel_tc(inputs, targets):
    z = jnp.concatenate([inputs.reshape(-1), targets.reshape(-1)]).reshape(R, C)
    out = pl.pallas_call(
        _chamfer_body,
        out_shape=jax.ShapeDtypeStruct((1, 1), jnp.float32),
    )(z)
    return out[0, 0]


import functools
from jax.experimental.pallas import tpu as pltpu
from jax.experimental.pallas import tpu_sc as plsc

NCORES = 1
NWS = 16
NVS = 64
LSC = 16
BLK_SC = NVS * LSC
MSC = NWS * BLK_SC


def _vg(ref, i):
    return ref[pl.ds(i * LSC, LSC)]


def _vs(ref, i, val):
    ref[pl.ds(i * LSC, LSC)] = val


def _clean_pairs(buf, h, d):
    @plsc.parallel_loop(0, NVS // 2, unroll=8)
    def _(pp):
        q = pp // h
        p = pp % h
        i = q * (2 * h) + (p // d) * (2 * d) + p % d
        a = _vg(buf, i)
        b = _vg(buf, i + d)
        _vs(buf, i, jnp.minimum(a, b))
        _vs(buf, i + d, jnp.maximum(a, b))


def _iota16():
    return lax.iota(jnp.int32, LSC)


def _vreg_sort_pass(buf):
    iota = _iota16()

    @plsc.parallel_loop(0, NVS, unroll=2)
    def _(i):
        base = i * LSC
        v = _vg(buf, i)
        for k in range(1, 5):
            kb = 1 << k
            d = kb >> 1
            while d >= 1:
                _vs(buf, i, v)
                b = plsc.load_gather(buf, [base + (iota ^ d)])
                wantmin = ((iota & d) == 0) == ((iota & kb) == 0)
                v = jnp.where(wantmin, jnp.minimum(v, b), jnp.maximum(v, b))
                d //= 2
        _vs(buf, i, v)


def _intra_clean_pass(buf):
    iota = _iota16()

    @plsc.parallel_loop(0, NVS, unroll=2)
    def _(i):
        base = i * LSC
        v = _vg(buf, i)
        for d in (8, 4, 2, 1):
            b = plsc.load_gather(buf, [base + (iota ^ d)])
            mask = (iota & d) == 0
            v = jnp.where(mask, jnp.minimum(v, b), jnp.maximum(v, b))
            _vs(buf, i, v)


def _rev_vreg(buf, i):
    return plsc.load_gather(buf, [i * LSC + (15 - _iota16())])


def _merge_first_stage(buf, h):
    iota = _iota16()
    rev = 15 - iota
    npair = max(h // 2, 1)

    @plsc.parallel_loop(0, (NVS // (2 * h)) * npair, unroll=4)
    def _(u):
        q = u // npair
        t = u % npair
        base = q * (2 * h)
        if h == 1:
            a = _vg(buf, base)
            b = plsc.load_gather(buf, [(base + 1) * LSC + rev])
            _vs(buf, base, jnp.minimum(a, b))
            _vs(buf, base + 1, jnp.maximum(a, b))
        else:
            t2 = h - 1 - t
            a1 = _vg(buf, base + t)
            a2 = _vg(buf, base + t2)
            b1 = plsc.load_gather(buf, [(base + 2 * h - 1 - t) * LSC + rev])
            b2 = plsc.load_gather(buf, [(base + h + t) * LSC + rev])
            _vs(buf, base + t, jnp.minimum(a1, b1))
            _vs(buf, base + t2, jnp.minimum(a2, b2))
            _vs(buf, base + h + t, jnp.maximum(a1, b1))
            _vs(buf, base + 2 * h - 1 - t, jnp.maximum(a2, b2))


def _local_reverse_second_halves(buf, rvb, h):
    @plsc.parallel_loop(0, NVS // 2, unroll=4)
    def _(u):
        q = u // h
        t = u % h
        _vs(rvb, q * h + (h - 1 - t), _rev_vreg(buf, q * (2 * h) + h + t))

    @plsc.parallel_loop(0, NVS // 2, unroll=4)
    def _(u):
        q = u // h
        t = u % h
        _vs(buf, q * (2 * h) + h + t, _vg(rvb, q * h + t))


NG = 8
HALF = NG * BLK_SC


def _sc_body(z_hbm, part_hbm, out_hbm, own, prt, rvb, oth, srow, sall, s0, s1):
    c = lax.axis_index("c")
    w = lax.axis_index("s")
    gid = w // NG
    gw = w % NG
    base = gid * NG
    coff = c * NWS

    pltpu.sync_copy(z_hbm.at[pl.ds(w * BLK_SC, BLK_SC)], own)

    _vreg_sort_pass(own)
    for m in range(1, 7):
        h = 1 << (m - 1)
        _merge_first_stage(own, h)
        d = h // 2
        while d >= 1:
            _clean_pairs(own, h, d)
            d //= 2
        _intra_clean_pass(own)

    bufs = (s0, s1)
    cur = 0
    for a_idx in range(1, 4):
        P = 1 << (a_idx - 1)
        plsc.subcore_barrier()
        g = gw // P
        q = gw % P
        is_second = (g % 2) == 1
        dest = base + g * P + (P - 1 - q)

        @plsc.parallel_loop(0, NVS, unroll=4)
        def _(u):
            _vs(rvb, u, _rev_vreg(own, NVS - 1 - u))

        @pl.when(is_second)
        def _():
            pltpu.sync_copy(
                rvb, bufs[cur].at[pl.ds((coff + dest) * BLK_SC, BLK_SC)]
            )

        @pl.when(jnp.logical_not(is_second))
        def _():
            pltpu.sync_copy(
                own, bufs[cur].at[pl.ds((coff + w) * BLK_SC, BLK_SC)]
            )

        plsc.subcore_barrier()

        @pl.when(is_second)
        def _():
            pltpu.sync_copy(
                bufs[cur].at[pl.ds((coff + w) * BLK_SC, BLK_SC)], own
            )

        dw = P
        while dw >= 1:
            pr = w ^ dw
            pltpu.sync_copy(
                bufs[cur].at[pl.ds((coff + pr) * BLK_SC, BLK_SC)], prt
            )
            wantmin = (gw & dw) == 0

            @plsc.parallel_loop(0, NVS, unroll=4)
            def _(i):
                a = _vg(own, i)
                b = _vg(prt, i)
                r = jnp.where(wantmin, jnp.minimum(a, b), jnp.maximum(a, b))
                _vs(own, i, r)
            if dw > 1:
                nxt = 1 - cur
                pltpu.sync_copy(
                    own, bufs[nxt].at[pl.ds((coff + w) * BLK_SC, BLK_SC)]
                )
                plsc.subcore_barrier()
                cur = nxt
            dw //= 2

        d = NVS // 2
        while d >= 1:
            _clean_pairs(own, NVS // 2, d)
            d //= 2
        _intra_clean_pass(own)

    plsc.subcore_barrier()
    pltpu.sync_copy(own, bufs[cur].at[pl.ds((coff + w) * BLK_SC, BLK_SC)])
    plsc.subcore_barrier()
    other_base = coff * BLK_SC + (1 - gid) * HALF
    pltpu.sync_copy(bufs[cur].at[pl.ds(other_base, HALF)], oth)

    neg = float("-inf")
    pos = float("inf")
    qv_per = NVS // NCORES
    lo_v = c * qv_per

    def srch(i, acc):
        qv = _vg(own, i)
        r = jnp.zeros((LSC,), jnp.int32)
        for e in range(12, -1, -1):
            sz = 1 << e
            idx = r + (sz - 1)
            v = plsc.load_gather(oth, [idx])
            r = jnp.where(v <= qv, r + sz, r)
        vtop = plsc.load_gather(oth, [r])
        r = jnp.where(vtop <= qv, r + 1, r)
        vp = plsc.load_gather(oth, [jnp.maximum(r - 1, 0)])
        pred = jnp.where(r > 0, vp, neg)
        succ = jnp.where(r < HALF, vtop, pos)
        dmin = jnp.minimum(qv - pred, succ - qv)
        return acc + dmin

    acc = plsc.parallel_loop(
        lo_v, lo_v + qv_per, unroll=4, carry=jnp.zeros((LSC,), jnp.float32)
    )(srch)

    srow[...] = acc
    pltpu.sync_copy(srow, part_hbm.at[c * NWS + w])
    plsc.subcore_barrier()

    @pl.when(w == 0)
    def _():
        pltpu.sync_copy(part_hbm.at[pl.ds(c * NWS, NWS)], sall)
        total = jnp.zeros((LSC,), jnp.float32)
        for j in range(NWS):
            total = total + sall[j]
        tsum = jnp.sum(total) * (0.5 / HALF)
        srow[...] = jnp.full((LSC,), tsum, dtype=jnp.float32)
        pltpu.sync_copy(srow, out_hbm.at[c])


def _kernel_sc(inputs, targets):
    z = jnp.concatenate([inputs.reshape(-1), targets.reshape(-1)])
    mesh = plsc.VectorSubcoreMesh(
        core_axis_name="c",
        subcore_axis_name="s",
        num_cores=NCORES,
        num_subcores=NWS,
    )
    run = pl.kernel(
        _sc_body,
        out_type=(
            jax.ShapeDtypeStruct((NCORES * NWS, LSC), jnp.float32),
            jax.ShapeDtypeStruct((NCORES, LSC), jnp.float32),
        ),
        mesh=mesh,
        compiler_params=pltpu.CompilerParams(needs_layout_passes=False),
        scratch_types=[
            pltpu.VMEM((BLK_SC,), jnp.float32),
            pltpu.VMEM((BLK_SC,), jnp.float32),
            pltpu.VMEM((BLK_SC,), jnp.float32),
            pltpu.VMEM((HALF,), jnp.float32),
            pltpu.VMEM((LSC,), jnp.float32),
            pltpu.VMEM((NWS, LSC), jnp.float32),
            pltpu.VMEM_SHARED((NCORES * MSC,), jnp.float32),
            pltpu.VMEM_SHARED((NCORES * MSC,), jnp.float32),
        ],
    )
    _, out = run(z)
    return jnp.sum(out[:, 0])


def kernel(inputs, targets):
    return _kernel_sc(inputs, targets)


# speedup vs baseline: 1.0281x; 1.0281x over previous
import jax
import jax.numpy as jnp
from jax import lax
from jax.experimental import pallas as pl
from jax.experimental.pallas import tpu as pltpu
from jax.experimental.pallas import tpu_sc as plsc

NCORES = 1
NWS = 16
NVS = 64
LSC = 16
BLK_SC = NVS * LSC
MSC = NWS * BLK_SC


def _vg(ref, i):
    return ref[pl.ds(i * LSC, LSC)]


def _vs(ref, i, val):
    ref[pl.ds(i * LSC, LSC)] = val


def _clean_pairs(buf, h, d):
    @plsc.parallel_loop(0, NVS // 2, unroll=4)
    def _(pp):
        q = pp // h
        p = pp % h
        i = q * (2 * h) + (p // d) * (2 * d) + p % d
        a = _vg(buf, i)
        b = _vg(buf, i + d)
        _vs(buf, i, jnp.minimum(a, b))
        _vs(buf, i + d, jnp.maximum(a, b))


def _iota16():
    return lax.iota(jnp.int32, LSC)


def _vreg_sort_pass(buf):
    iota = _iota16()

    @plsc.parallel_loop(0, NVS, unroll=2)
    def _(i):
        base = i * LSC
        v = _vg(buf, i)
        for k in range(1, 5):
            kb = 1 << k
            d = kb >> 1
            while d >= 1:
                _vs(buf, i, v)
                b = plsc.load_gather(buf, [base + (iota ^ d)])
                wantmin = ((iota & d) == 0) == ((iota & kb) == 0)
                v = jnp.where(wantmin, jnp.minimum(v, b), jnp.maximum(v, b))
                d //= 2
        _vs(buf, i, v)


def _intra_clean_pass(buf):
    iota = _iota16()

    @plsc.parallel_loop(0, NVS, unroll=2)
    def _(i):
        base = i * LSC
        v = _vg(buf, i)
        for d in (8, 4, 2, 1):
            b = plsc.load_gather(buf, [base + (iota ^ d)])
            mask = (iota & d) == 0
            v = jnp.where(mask, jnp.minimum(v, b), jnp.maximum(v, b))
            _vs(buf, i, v)


def _rev_vreg(buf, i):
    return plsc.load_gather(buf, [i * LSC + (15 - _iota16())])


def _merge_first_stage(buf, h):
    iota = _iota16()
    rev = 15 - iota
    npair = max(h // 2, 1)

    @plsc.parallel_loop(0, (NVS // (2 * h)) * npair, unroll=2)
    def _(u):
        q = u // npair
        t = u % npair
        base = q * (2 * h)
        if h == 1:
            a = _vg(buf, base)
            b = plsc.load_gather(buf, [(base + 1) * LSC + rev])
            _vs(buf, base, jnp.minimum(a, b))
            _vs(buf, base + 1, jnp.maximum(a, b))
        else:
            t2 = h - 1 - t
            a1 = _vg(buf, base + t)
            a2 = _vg(buf, base + t2)
            b1 = plsc.load_gather(buf, [(base + 2 * h - 1 - t) * LSC + rev])
            b2 = plsc.load_gather(buf, [(base + h + t) * LSC + rev])
            _vs(buf, base + t, jnp.minimum(a1, b1))
            _vs(buf, base + t2, jnp.minimum(a2, b2))
            _vs(buf, base + h + t, jnp.maximum(a1, b1))
            _vs(buf, base + 2 * h - 1 - t, jnp.maximum(a2, b2))


NG = 8
HALF = NG * BLK_SC


def _sc_body(z_hbm, part_hbm, out_hbm, own, prt, rvb, oth, srow, sall, s0, s1):
    c = lax.axis_index("c")
    w = lax.axis_index("s")
    gid = w // NG
    gw = w % NG
    base = gid * NG
    coff = c * NWS

    pltpu.sync_copy(z_hbm.at[pl.ds(w * BLK_SC, BLK_SC)], own)

    _vreg_sort_pass(own)
    for m in range(1, 7):
        h = 1 << (m - 1)
        _merge_first_stage(own, h)
        d = h // 2
        while d >= 1:
            _clean_pairs(own, h, d)
            d //= 2
        _intra_clean_pass(own)

    bufs = (s0, s1)
    cur = 0
    for a_idx in range(1, 4):
        P = 1 << (a_idx - 1)
        plsc.subcore_barrier()
        g = gw // P
        q = gw % P
        is_second = (g % 2) == 1
        dest = base + g * P + (P - 1 - q)

        @plsc.parallel_loop(0, NVS, unroll=4)
        def _(u):
            _vs(rvb, u, _rev_vreg(own, NVS - 1 - u))

        @pl.when(is_second)
        def _():
            pltpu.sync_copy(
                rvb, bufs[cur].at[pl.ds((coff + dest) * BLK_SC, BLK_SC)]
            )

        @pl.when(jnp.logical_not(is_second))
        def _():
            pltpu.sync_copy(
                own, bufs[cur].at[pl.ds((coff + w) * BLK_SC, BLK_SC)]
            )

        plsc.subcore_barrier()

        @pl.when(is_second)
        def _():
            pltpu.sync_copy(
                bufs[cur].at[pl.ds((coff + w) * BLK_SC, BLK_SC)], own
            )

        dw = P
        while dw >= 1:
            pr = w ^ dw
            pltpu.sync_copy(
                bufs[cur].at[pl.ds((coff + pr) * BLK_SC, BLK_SC)], prt
            )
            wantmin = (gw & dw) == 0

            @plsc.parallel_loop(0, NVS, unroll=4)
            def _(i):
                a = _vg(own, i)
                b = _vg(prt, i)
                r = jnp.where(wantmin, jnp.minimum(a, b), jnp.maximum(a, b))
                _vs(own, i, r)
            if dw > 1:
                nxt = 1 - cur
                pltpu.sync_copy(
                    own, bufs[nxt].at[pl.ds((coff + w) * BLK_SC, BLK_SC)]
                )
                plsc.subcore_barrier()
                cur = nxt
            dw //= 2

        d = NVS // 2
        while d >= 1:
            _clean_pairs(own, NVS // 2, d)
            d //= 2
        _intra_clean_pass(own)

    plsc.subcore_barrier()
    pltpu.sync_copy(own, bufs[cur].at[pl.ds((coff + w) * BLK_SC, BLK_SC)])
    plsc.subcore_barrier()
    other_base = coff * BLK_SC + (1 - gid) * HALF
    pltpu.sync_copy(bufs[cur].at[pl.ds(other_base, HALF)], oth)

    neg = float("-inf")
    pos = float("inf")
    qv_per = NVS // NCORES
    lo_v = c * qv_per

    def srch(i, acc):
        qv = _vg(own, i)
        r = jnp.zeros((LSC,), jnp.int32)
        for e in range(12, -1, -1):
            sz = 1 << e
            idx = r + (sz - 1)
            v = plsc.load_gather(oth, [idx])
            r = jnp.where(v <= qv, r + sz, r)
        vtop = plsc.load_gather(oth, [r])
        r = jnp.where(vtop <= qv, r + 1, r)
        vp = plsc.load_gather(oth, [jnp.maximum(r - 1, 0)])
        pred = jnp.where(r > 0, vp, neg)
        succ = jnp.where(r < HALF, vtop, pos)
        dmin = jnp.minimum(qv - pred, succ - qv)
        return acc + dmin

    acc = plsc.parallel_loop(
        lo_v, lo_v + qv_per, unroll=4, carry=jnp.zeros((LSC,), jnp.float32)
    )(srch)

    srow[...] = acc
    pltpu.sync_copy(srow, part_hbm.at[c * NWS + w])
    plsc.subcore_barrier()

    @pl.when(w == 0)
    def _():
        pltpu.sync_copy(part_hbm.at[pl.ds(c * NWS, NWS)], sall)
        total = jnp.zeros((LSC,), jnp.float32)
        for j in range(NWS):
            total = total + sall[j]
        tsum = jnp.sum(total) * (0.5 / HALF)
        srow[...] = jnp.full((LSC,), tsum, dtype=jnp.float32)
        pltpu.sync_copy(srow, out_hbm.at[c])


def _kernel_sc(inputs, targets):
    z = jnp.concatenate([inputs.reshape(-1), targets.reshape(-1)])
    mesh = plsc.VectorSubcoreMesh(
        core_axis_name="c",
        subcore_axis_name="s",
        num_cores=NCORES,
        num_subcores=NWS,
    )
    run = pl.kernel(
        _sc_body,
        out_type=(
            jax.ShapeDtypeStruct((NCORES * NWS, LSC), jnp.float32),
            jax.ShapeDtypeStruct((NCORES, LSC), jnp.float32),
        ),
        mesh=mesh,
        compiler_params=pltpu.CompilerParams(needs_layout_passes=False),
        scratch_types=[
            pltpu.VMEM((BLK_SC,), jnp.float32),
            pltpu.VMEM((BLK_SC,), jnp.float32),
            pltpu.VMEM((BLK_SC,), jnp.float32),
            pltpu.VMEM((HALF,), jnp.float32),
            pltpu.VMEM((LSC,), jnp.float32),
            pltpu.VMEM((NWS, LSC), jnp.float32),
            pltpu.VMEM_SHARED((NCORES * MSC,), jnp.float32),
            pltpu.VMEM_SHARED((NCORES * MSC,), jnp.float32),
        ],
    )
    _, out = run(z)
    return jnp.sum(out[:, 0])


def kernel(inputs, targets):
    return _kernel_sc(inputs, targets)
